# baseline (speedup 1.0000x reference)
import jax
import jax.numpy as jnp
from jax.experimental import pallas as pl

K = 16
EPS = 1e-5


def _copy_kernel(x_ref, o_ref):
    o_ref[...] = x_ref[...]


def _bn(x, g, b, axes):
    m = jnp.mean(x, axis=axes, keepdims=True)
    v = jnp.var(x, axis=axes, keepdims=True)
    sh = [1] * x.ndim
    sh[1] = -1
    return (x - m) / jnp.sqrt(v + EPS) * g.reshape(sh) + b.reshape(sh)


def kernel(xyz, new_xyz, feature, new_feature, wn_w1, wn_g1, wn_b1, wn_w2, wn_g2, wn_b2, wn_w3, wn_g3, wn_b3, fin_w, fin_g, fin_b):
    B, N, _ = xyz.shape
    d = (jnp.sum(xyz * xyz, axis=-1, keepdims=True)
         + jnp.sum(new_xyz * new_xyz, axis=-1)[:, None, :]
         - 2.0 * jnp.einsum('bnd,bmd->bnm', xyz, new_xyz))
    _, idx = jax.lax.top_k(-d, K)
    grouped_new_xyz = jax.vmap(lambda f, i: f[:, i])(jnp.transpose(new_xyz, (0, 2, 1)), idx)
    trans = grouped_new_xyz - jnp.transpose(xyz, (0, 2, 1))[..., None]
    gnf = jax.vmap(lambda f, i: f[:, i])(new_feature, idx)
    w = trans
    for ww, gg, bb in ((wn_w1, wn_g1, wn_b1), (wn_w2, wn_g2, wn_b2), (wn_w3, wn_g3, wn_b3)):
        w = jnp.einsum('oc,bcnk->bonk', ww, w)
        w = jax.nn.relu(_bn(w, gg, bb, (0, 2, 3)))
    nf = jnp.matmul(jnp.transpose(gnf, (0, 2, 1, 3)), jnp.transpose(w, (0, 2, 3, 1)))
    nf = jnp.transpose(nf.reshape(B, N, -1), (0, 2, 1))
    cat = jnp.concatenate([nf, feature], axis=1)
    y = jnp.einsum('oc,bcn->bon', fin_w, cat)
    y = jax.nn.relu(_bn(y, fin_g, fin_b, (0, 2)))
    y = pl.pallas_call(
        _copy_kernel,
        out_shape=jax.ShapeDtypeStruct(y.shape, y.dtype),
    )(y)
    return y


# speedup vs baseline: 1.6315x; 1.6315x over previous
import functools

import jax
import jax.numpy as jnp
from jax.experimental import pallas as pl

K = 16
EPS = 1e-5
TN = 256


def _knn_body(xyz_ref, nxyz_t_ref, idx_ref, *, m):
    x = xyz_ref[0]
    yt = nxyz_t_ref[0]
    xx = jnp.sum(x * x, axis=1, keepdims=True)
    yy = jnp.sum(yt * yt, axis=0, keepdims=True)
    d = xx + yy - 2.0 * jax.lax.dot(x, yt, preferred_element_type=jnp.float32)
    lane = jax.lax.broadcasted_iota(jnp.int32, (TN, m), 1)
    cols = []
    for _ in range(K):
        mn = jnp.min(d, axis=1, keepdims=True)
        am = jnp.min(jnp.where(d == mn, lane, m), axis=1)
        cols.append(am)
        d = jnp.where(lane == am[:, None], jnp.inf, d)
    idx_ref[0] = jnp.stack(cols, axis=1)


def _knn(xyz, new_xyz_t):
    b, n, _ = xyz.shape
    m = new_xyz_t.shape[2]
    grid = (b, n // TN)
    return pl.pallas_call(
        functools.partial(_knn_body, m=m),
        grid=grid,
        in_specs=[
            pl.BlockSpec((1, TN, 3), lambda i, j: (i, j, 0)),
            pl.BlockSpec((1, 3, m), lambda i, j: (i, 0, 0)),
        ],
        out_specs=pl.BlockSpec((1, TN, K), lambda i, j: (i, j, 0)),
        out_shape=jax.ShapeDtypeStruct((b, n, K), jnp.int32),
    )(xyz, new_xyz_t)


def _bn(x, g, b, axes):
    mu = jnp.mean(x, axis=axes, keepdims=True)
    v = jnp.var(x, axis=axes, keepdims=True)
    sh = [1] * x.ndim
    sh[1] = -1
    return (x - mu) / jnp.sqrt(v + EPS) * g.reshape(sh) + b.reshape(sh)


def kernel(xyz, new_xyz, feature, new_feature, wn_w1, wn_g1, wn_b1, wn_w2, wn_g2, wn_b2, wn_w3, wn_g3, wn_b3, fin_w, fin_g, fin_b):
    B, N, _ = xyz.shape
    new_xyz_t = jnp.transpose(new_xyz, (0, 2, 1))
    idx = _knn(xyz, new_xyz_t)
    grouped_new_xyz = jax.vmap(lambda f, i: f[:, i])(new_xyz_t, idx)
    trans = grouped_new_xyz - jnp.transpose(xyz, (0, 2, 1))[..., None]
    gnf = jax.vmap(lambda f, i: f[:, i])(new_feature, idx)
    w = trans
    for ww, gg, bb in ((wn_w1, wn_g1, wn_b1), (wn_w2, wn_g2, wn_b2), (wn_w3, wn_g3, wn_b3)):
        w = jnp.einsum('oc,bcnk->bonk', ww, w)
        w = jax.nn.relu(_bn(w, gg, bb, (0, 2, 3)))
    nf = jnp.matmul(jnp.transpose(gnf, (0, 2, 1, 3)), jnp.transpose(w, (0, 2, 3, 1)))
    nf = jnp.transpose(nf.reshape(B, N, -1), (0, 2, 1))
    cat = jnp.concatenate([nf, feature], axis=1)
    y = jnp.einsum('oc,bcn->bon', fin_w, cat)
    y = jax.nn.relu(_bn(y, fin_g, fin_b, (0, 2)))
    return y


# speedup vs baseline: 14.7302x; 9.0287x over previous
import functools

import jax
import jax.numpy as jnp
from jax import lax
from jax.experimental import pallas as pl
from jax.experimental.pallas import tpu as pltpu
from jax.experimental.pallas import tpu_sc as plsc

K = 16
EPS = 1e-5
TN = 256
TN2 = 512
NCHUNK = 512
_NC, _NS = 2, 16


def _knn_body(xyz_ref, nxyz_t_ref, idx_ref, *, m):
    x = xyz_ref[0]
    yt = nxyz_t_ref[0]
    xx = jnp.sum(x * x, axis=1, keepdims=True)
    yy = jnp.sum(yt * yt, axis=0, keepdims=True)
    d = xx + yy - 2.0 * jax.lax.dot(x, yt, preferred_element_type=jnp.float32)
    lane = jax.lax.broadcasted_iota(jnp.int32, (TN, m), 1)
    cols = []
    for _ in range(K):
        mn = jnp.min(d, axis=1, keepdims=True)
        am = jnp.min(jnp.where(d == mn, lane, m), axis=1)
        cols.append(am)
        d = jnp.where(lane == am[:, None], jnp.inf, d)
    idx_ref[0] = jnp.stack(cols, axis=0).reshape(K // 2, 2, TN)


def _knn(xyz, new_xyz_t):
    b, n, _ = xyz.shape
    m = new_xyz_t.shape[2]
    return pl.pallas_call(
        functools.partial(_knn_body, m=m),
        grid=(b, n // TN),
        in_specs=[
            pl.BlockSpec((1, TN, 3), lambda i, j: (i, j, 0)),
            pl.BlockSpec((1, 3, m), lambda i, j: (i, 0, 0)),
        ],
        out_specs=pl.BlockSpec((1, K // 2, 2, TN), lambda i, j: (i, 0, 0, j)),
        out_shape=jax.ShapeDtypeStruct((b, K // 2, 2, n), jnp.int32),
    )(xyz, new_xyz_t)


def _gather_body(tbl_hbm, idx_hbm, gx_hbm, gf_hbm, tbl_v, idx_v, fbuf_v, xbuf_v, *, n, m):
    cid = lax.axis_index("c")
    sid = lax.axis_index("s")
    wid = sid * _NC + cid
    b = wid // 8
    kp = wid % 8
    pltpu.sync_copy(tbl_hbm.at[b], tbl_v)
    lane = lax.iota(jnp.int32, 16)

    def chunk_body(ci, carry):
        n0 = ci * NCHUNK
        pltpu.sync_copy(idx_hbm.at[b, kp, :, pl.ds(n0, NCHUNK)], idx_v)
        for kk in range(2):
            k = kp * 2 + kk

            def grp_body(g, carry2):
                sl = pl.ds(g * 16, 16)
                i16 = idx_v[kk, sl]
                for cch in range(35):
                    v = plsc.load_gather(tbl_v, [i16 + cch * m])
                    if cch < 3:
                        xbuf_v[kk * 4 + cch, sl] = v
                    else:
                        fbuf_v[cch - 3, sl] = v
                        if cch == 3:
                            xbuf_v[kk * 4 + 3, sl] = v
                return carry2

            lax.fori_loop(0, NCHUNK // 16, grp_body, 0)
            pltpu.sync_copy(fbuf_v, gf_hbm.at[b, pl.ds(k * 32, 32), pl.ds(n0, NCHUNK)])
        pltpu.sync_copy(xbuf_v, gx_hbm.at[b, pl.ds(kp * 8, 8), pl.ds(n0, NCHUNK)])
        return carry

    lax.fori_loop(0, n // NCHUNK, chunk_body, 0)


def _gather(tbl_flat, idx, m):
    b = tbl_flat.shape[0]
    n = idx.shape[3]
    mesh = plsc.VectorSubcoreMesh(core_axis_name="c", subcore_axis_name="s")
    return pl.kernel(
        functools.partial(_gather_body, n=n, m=m),
        out_type=(
            jax.ShapeDtypeStruct((b, 4 * K, n), jnp.float32),
            jax.ShapeDtypeStruct((b, 32 * K, n), jnp.float32),
        ),
        mesh=mesh,
        scratch_types=(
            pltpu.VMEM((tbl_flat.shape[1],), jnp.float32),
            pltpu.VMEM((2, NCHUNK), jnp.int32),
            pltpu.VMEM((32, NCHUNK), jnp.float32),
            pltpu.VMEM((8, NCHUNK), jnp.float32),
        ),
        compiler_params=pltpu.CompilerParams(needs_layout_passes=False),
    )(tbl_flat, idx)


def _dot3(a, b):
    ah = a.astype(jnp.bfloat16)
    al = (a - ah.astype(jnp.float32)).astype(jnp.bfloat16)
    bh = b.astype(jnp.bfloat16)
    bl = (b - bh.astype(jnp.float32)).astype(jnp.bfloat16)
    f = functools.partial(jax.lax.dot, preferred_element_type=jnp.float32)
    return f(ah, bh) + (f(ah, bl) + f(al, bh))


def _mlp(tr, wts):
    h = tr
    for i, (wk, bias) in enumerate(wts):
        h = _dot3(wk, h)
        if bias is not None:
            h = jnp.maximum(h + bias, 0.0)
    return h


def _accum(ref, part, first):
    @pl.when(first)
    def _():
        ref[...] = jnp.zeros_like(ref)
    ref[...] += part


def _stats_body(gx_ref, xyz48_ref, *rest, n_weights):
    wrefs = rest[:n_weights]
    st_ref = rest[n_weights]
    first = (pl.program_id(0) == 0) & (pl.program_id(1) == 0)
    tr = gx_ref[0] - xyz48_ref[0]
    wts = []
    i = 0
    while i < len(wrefs):
        if i + 1 < len(wrefs):
            wts.append((wrefs[i][...], wrefs[i + 1][...]))
            i += 2
        else:
            wts.append((wrefs[i][...], None))
            i += 1
    r = _mlp(tr, wts)
    part = jnp.stack([jnp.sum(r, axis=1), jnp.sum(r * r, axis=1)], axis=0)
    _accum(st_ref, part, first)


def _stats_pass(gx, xyz48, weights):
    b, _, n = gx.shape
    wspecs = [pl.BlockSpec(w.shape, lambda i, j, nd=len(w.shape): (0,) * nd) for w in weights]
    return pl.pallas_call(
        functools.partial(_stats_body, n_weights=len(weights)),
        grid=(b, n // TN2),
        in_specs=[
            pl.BlockSpec((1, 64, TN2), lambda i, j: (i, 0, j)),
            pl.BlockSpec((1, 64, TN2), lambda i, j: (i, 0, j)),
        ] + wspecs,
        out_specs=pl.BlockSpec((2, 16 * K), lambda i, j: (0, 0)),
        out_shape=jax.ShapeDtypeStruct((2, 16 * K), jnp.float32),
    )(gx, xyz48, *weights)


def _main_body(gx_ref, xyz48_ref, gf_ref, feat_ref, w1_ref, t1_ref, w2_ref,
               t2_ref, w3_ref, t3_ref, fw_ref, y_ref, st_ref):
    first = (pl.program_id(0) == 0) & (pl.program_id(1) == 0)
    tr = gx_ref[0] - xyz48_ref[0]
    w = _mlp(tr, [(w1_ref[...], t1_ref[...]),
                  (w2_ref[...], t2_ref[...]),
                  (w3_ref[...], t3_ref[...])])
    w = jnp.maximum(w, 0.0)
    gf = gf_ref[0]
    parts = []
    for o in range(16):
        acc = None
        for k in range(K):
            p = gf[k * 32:(k + 1) * 32, :] * w[k * 16 + o:k * 16 + o + 1, :]
            acc = p if acc is None else acc + p
        parts.append(acc)
    cat = jnp.concatenate(parts + [feat_ref[0]], axis=0)
    y = _dot3(fw_ref[...], cat)
    y_ref[0] = y
    part = jnp.stack([jnp.sum(y, axis=1), jnp.sum(y * y, axis=1)], axis=0)
    _accum(st_ref, part, first)


def _main_pass(gx, xyz48, gf, feat, w1, t1, w2, t2, w3, t3, fw):
    b, _, n = gx.shape
    small = [w1, t1, w2, t2, w3, t3, fw]
    sspecs = [pl.BlockSpec(w.shape, lambda i, j, nd=len(w.shape): (0,) * nd) for w in small]
    return pl.pallas_call(
        _main_body,
        grid=(b, n // TN2),
        in_specs=[
            pl.BlockSpec((1, 64, TN2), lambda i, j: (i, 0, j)),
            pl.BlockSpec((1, 64, TN2), lambda i, j: (i, 0, j)),
            pl.BlockSpec((1, 512, TN2), lambda i, j: (i, 0, j)),
            pl.BlockSpec((1, 64, TN2), lambda i, j: (i, 0, j)),
        ] + sspecs,
        out_specs=(
            pl.BlockSpec((1, 64, TN2), lambda i, j: (i, 0, j)),
            pl.BlockSpec((2, 64), lambda i, j: (0, 0)),
        ),
        out_shape=(
            jax.ShapeDtypeStruct((b, 64, n), jnp.float32),
            jax.ShapeDtypeStruct((2, 64), jnp.float32),
        ),
    )(gx, xyz48, gf, feat, *small)


def _final_body(y_ref, sc_ref, sh_ref, o_ref):
    o_ref[0] = jnp.maximum(y_ref[0] * sc_ref[...] + sh_ref[...], 0.0)


def _final_pass(y_raw, scale, shift):
    b, c, n = y_raw.shape
    return pl.pallas_call(
        _final_body,
        grid=(b, n // TN2),
        in_specs=[
            pl.BlockSpec((1, c, TN2), lambda i, j: (i, 0, j)),
            pl.BlockSpec((c, 1), lambda i, j: (0, 0)),
            pl.BlockSpec((c, 1), lambda i, j: (0, 0)),
        ],
        out_specs=pl.BlockSpec((1, c, TN2), lambda i, j: (i, 0, j)),
        out_shape=jax.ShapeDtypeStruct((b, c, n), jnp.float32),
    )(y_raw, scale, shift)


def _fold_stats(part, cnt, g, bias):
    c = part.shape[1] // K
    sums = part.reshape(2, K, c).sum(axis=1)
    mean = sums[0] / cnt
    var = sums[1] / cnt - mean * mean
    scale = g / jnp.sqrt(var + EPS)
    shift = bias - mean * scale
    return jnp.tile(scale, K).reshape(-1, 1), jnp.tile(shift, K).reshape(-1, 1)


def kernel(xyz, new_xyz, feature, new_feature, wn_w1, wn_g1, wn_b1, wn_w2,
           wn_g2, wn_b2, wn_w3, wn_g3, wn_b3, fin_w, fin_g, fin_b):
    B, N, _ = xyz.shape
    new_xyz_t = jnp.transpose(new_xyz, (0, 2, 1))
    xyz_t = jnp.transpose(xyz, (0, 2, 1))
    xyzp = jnp.concatenate([xyz_t, jnp.zeros((B, 1, N), jnp.float32)], axis=1)
    xyz48 = jnp.tile(xyzp, (1, K, 1))
    tbl = jnp.concatenate([new_xyz_t, new_feature], axis=1)

    idx = _knn(xyz, new_xyz_t)
    gx, gf = _gather(tbl.reshape(B, -1), idx, tbl.shape[2])

    eye = jnp.eye(K, dtype=jnp.float32)
    w1p = jnp.concatenate([wn_w1, jnp.zeros((16, 1), jnp.float32)], axis=1)
    w1k = jnp.kron(eye, w1p)
    w2k = jnp.kron(eye, wn_w2)
    w3k = jnp.kron(eye, wn_w3)

    cnt = jnp.float32(B * N * K)
    s1 = _stats_pass(gx, xyz48, [w1k])
    sc1, t1 = _fold_stats(s1, cnt, wn_g1, wn_b1)
    w1s = w1k * sc1
    s2 = _stats_pass(gx, xyz48, [w1s, t1, w2k])
    sc2, t2 = _fold_stats(s2, cnt, wn_g2, wn_b2)
    w2s = w2k * sc2
    s3 = _stats_pass(gx, xyz48, [w1s, t1, w2s, t2, w3k])
    sc3, t3 = _fold_stats(s3, cnt, wn_g3, wn_b3)
    w3s = w3k * sc3

    perm = jnp.asarray([(r % 32) * K + r // 32 for r in range(512)], jnp.int32)
    fwp = jnp.concatenate([fin_w[:, :512][:, perm], fin_w[:, 512:]], axis=1)

    y_raw, sy = _main_pass(gx, xyz48, gf, feature, w1s, t1, w2s, t2, w3s, t3, fwp)
    cnt_y = jnp.float32(B * N)
    mean_y = sy[0] / cnt_y
    var_y = sy[1] / cnt_y - mean_y * mean_y
    fsc = (fin_g / jnp.sqrt(var_y + EPS)).reshape(-1, 1)
    fsh = (fin_b - mean_y * fsc[:, 0]).reshape(-1, 1)
    return _final_pass(y_raw, fsc, fsh)


# speedup vs baseline: 15.9910x; 1.0856x over previous
import functools

import jax
import jax.numpy as jnp
from jax import lax
from jax.experimental import pallas as pl
from jax.experimental.pallas import tpu as pltpu
from jax.experimental.pallas import tpu_sc as plsc

K = 16
EPS = 1e-5
TN = 512
TN2 = 512
NCHUNK = 512
_NC, _NS = 2, 16


def _knn_body(xyz_ref, nxyz_t_ref, idx_ref, *, m):
    x = xyz_ref[0]
    yt = nxyz_t_ref[0]
    xx = jnp.sum(x * x, axis=1, keepdims=True)
    yy = jnp.sum(yt * yt, axis=0, keepdims=True)
    d = xx + yy - 2.0 * jax.lax.dot(x, yt, preferred_element_type=jnp.float32)
    lane = jax.lax.broadcasted_iota(jnp.int32, (TN, m), 1)
    cols = []
    for _ in range(K):
        mn = jnp.min(d, axis=1, keepdims=True)
        am = jnp.min(jnp.where(d == mn, lane, m), axis=1)
        cols.append(am)
        d = jnp.where(lane == am[:, None], jnp.inf, d)
    idx_ref[0] = jnp.stack(cols, axis=0).reshape(K // 2, 2, TN)


def _knn(xyz, new_xyz_t):
    b, n, _ = xyz.shape
    m = new_xyz_t.shape[2]
    return pl.pallas_call(
        functools.partial(_knn_body, m=m),
        grid=(b, n // TN),
        in_specs=[
            pl.BlockSpec((1, TN, 3), lambda i, j: (i, j, 0)),
            pl.BlockSpec((1, 3, m), lambda i, j: (i, 0, 0)),
        ],
        out_specs=pl.BlockSpec((1, K // 2, 2, TN), lambda i, j: (i, 0, 0, j)),
        out_shape=jax.ShapeDtypeStruct((b, K // 2, 2, n), jnp.int32),
    )(xyz, new_xyz_t)


def _gather_body(tbl_hbm, idx_hbm, gx_hbm, gf_hbm, tbl_v, idx_v, fbuf_v, xbuf_v, *, n, m):
    cid = lax.axis_index("c")
    sid = lax.axis_index("s")
    wid = sid * _NC + cid
    b = wid // 8
    kp = wid % 8
    pltpu.sync_copy(tbl_hbm.at[b], tbl_v)
    lane = lax.iota(jnp.int32, 16)

    def chunk_body(ci, carry):
        n0 = ci * NCHUNK
        pltpu.sync_copy(idx_hbm.at[b, kp, :, pl.ds(n0, NCHUNK)], idx_v)
        for kk in range(2):
            k = kp * 2 + kk

            def grp_body(g, carry2):
                sl = pl.ds(g * 16, 16)
                i16 = idx_v[kk, sl]
                for cch in range(35):
                    v = plsc.load_gather(tbl_v, [i16 + cch * m])
                    if cch < 3:
                        xbuf_v[kk * 4 + cch, sl] = v
                    else:
                        fbuf_v[cch - 3, sl] = v
                        if cch == 3:
                            xbuf_v[kk * 4 + 3, sl] = v
                return carry2

            lax.fori_loop(0, NCHUNK // 16, grp_body, 0)
            pltpu.sync_copy(fbuf_v, gf_hbm.at[b, pl.ds(k * 32, 32), pl.ds(n0, NCHUNK)])
        pltpu.sync_copy(xbuf_v, gx_hbm.at[b, pl.ds(kp * 8, 8), pl.ds(n0, NCHUNK)])
        return carry

    lax.fori_loop(0, n // NCHUNK, chunk_body, 0)


def _gather(tbl_flat, idx, m):
    b = tbl_flat.shape[0]
    n = idx.shape[3]
    mesh = plsc.VectorSubcoreMesh(core_axis_name="c", subcore_axis_name="s")
    return pl.kernel(
        functools.partial(_gather_body, n=n, m=m),
        out_type=(
            jax.ShapeDtypeStruct((b, 4 * K, n), jnp.float32),
            jax.ShapeDtypeStruct((b, 32 * K, n), jnp.float32),
        ),
        mesh=mesh,
        scratch_types=(
            pltpu.VMEM((tbl_flat.shape[1],), jnp.float32),
            pltpu.VMEM((2, NCHUNK), jnp.int32),
            pltpu.VMEM((32, NCHUNK), jnp.float32),
            pltpu.VMEM((8, NCHUNK), jnp.float32),
        ),
        compiler_params=pltpu.CompilerParams(needs_layout_passes=False),
    )(tbl_flat, idx)


def _dot3(a, b):
    ah = a.astype(jnp.bfloat16)
    al = (a - ah.astype(jnp.float32)).astype(jnp.bfloat16)
    bh = b.astype(jnp.bfloat16)
    bl = (b - bh.astype(jnp.float32)).astype(jnp.bfloat16)
    f = functools.partial(jax.lax.dot, preferred_element_type=jnp.float32)
    return f(ah, bh) + (f(ah, bl) + f(al, bh))


def _mlp(tr, wts, precise=True):
    h = tr
    for i, (wk, bias) in enumerate(wts):
        if precise:
            h = _dot3(wk, h)
        else:
            h = jax.lax.dot(wk, h, preferred_element_type=jnp.float32)
        if bias is not None:
            h = jnp.maximum(h + bias, 0.0)
    return h


def _accum(ref, part, first):
    @pl.when(first)
    def _():
        ref[...] = jnp.zeros_like(ref)
    ref[...] += part


def _stats_body(gx_ref, xyz48_ref, *rest, n_weights):
    wrefs = rest[:n_weights]
    st_ref = rest[n_weights]
    first = (pl.program_id(0) == 0) & (pl.program_id(1) == 0)
    tr = gx_ref[0] - xyz48_ref[0]
    wts = []
    i = 0
    while i < len(wrefs):
        if i + 1 < len(wrefs):
            wts.append((wrefs[i][...], wrefs[i + 1][...]))
            i += 2
        else:
            wts.append((wrefs[i][...], None))
            i += 1
    r = _mlp(tr, wts, precise=False)
    part = jnp.stack([jnp.sum(r, axis=1), jnp.sum(r * r, axis=1)], axis=0)
    _accum(st_ref, part, first)


def _stats_pass(gx, xyz48, weights):
    b, _, n = gx.shape
    wspecs = [pl.BlockSpec(w.shape, lambda i, j, nd=len(w.shape): (0,) * nd) for w in weights]
    return pl.pallas_call(
        functools.partial(_stats_body, n_weights=len(weights)),
        grid=(b, n // TN2),
        in_specs=[
            pl.BlockSpec((1, 64, TN2), lambda i, j: (i, 0, j)),
            pl.BlockSpec((1, 64, TN2), lambda i, j: (i, 0, j)),
        ] + wspecs,
        out_specs=pl.BlockSpec((2, 16 * K), lambda i, j: (0, 0)),
        out_shape=jax.ShapeDtypeStruct((2, 16 * K), jnp.float32),
    )(gx, xyz48, *weights)


def _main_body(gx_ref, xyz48_ref, gf_ref, feat_ref, w1_ref, t1_ref, w2_ref,
               t2_ref, w3_ref, t3_ref, fw_ref, y_ref, st_ref):
    first = (pl.program_id(0) == 0) & (pl.program_id(1) == 0)
    tr = gx_ref[0] - xyz48_ref[0]
    w = _mlp(tr, [(w1_ref[...], t1_ref[...]),
                  (w2_ref[...], t2_ref[...]),
                  (w3_ref[...], t3_ref[...])])
    w = jnp.maximum(w, 0.0)
    gf = gf_ref[0]
    parts = []
    for o in range(16):
        acc = None
        for k in range(K):
            p = gf[k * 32:(k + 1) * 32, :] * w[k * 16 + o:k * 16 + o + 1, :]
            acc = p if acc is None else acc + p
        parts.append(acc)
    cat = jnp.concatenate(parts + [feat_ref[0]], axis=0)
    y = _dot3(fw_ref[...], cat)
    y_ref[0] = y
    part = jnp.stack([jnp.sum(y, axis=1), jnp.sum(y * y, axis=1)], axis=0)
    _accum(st_ref, part, first)


def _main_pass(gx, xyz48, gf, feat, w1, t1, w2, t2, w3, t3, fw):
    b, _, n = gx.shape
    small = [w1, t1, w2, t2, w3, t3, fw]
    sspecs = [pl.BlockSpec(w.shape, lambda i, j, nd=len(w.shape): (0,) * nd) for w in small]
    return pl.pallas_call(
        _main_body,
        grid=(b, n // TN2),
        in_specs=[
            pl.BlockSpec((1, 64, TN2), lambda i, j: (i, 0, j)),
            pl.BlockSpec((1, 64, TN2), lambda i, j: (i, 0, j)),
            pl.BlockSpec((1, 512, TN2), lambda i, j: (i, 0, j)),
            pl.BlockSpec((1, 64, TN2), lambda i, j: (i, 0, j)),
        ] + sspecs,
        out_specs=(
            pl.BlockSpec((1, 64, TN2), lambda i, j: (i, 0, j)),
            pl.BlockSpec((2, 64), lambda i, j: (0, 0)),
        ),
        out_shape=(
            jax.ShapeDtypeStruct((b, 64, n), jnp.float32),
            jax.ShapeDtypeStruct((2, 64), jnp.float32),
        ),
    )(gx, xyz48, gf, feat, *small)


def _final_body(y_ref, sc_ref, sh_ref, o_ref):
    o_ref[0] = jnp.maximum(y_ref[0] * sc_ref[...] + sh_ref[...], 0.0)


def _final_pass(y_raw, scale, shift):
    b, c, n = y_raw.shape
    return pl.pallas_call(
        _final_body,
        grid=(b, n // TN2),
        in_specs=[
            pl.BlockSpec((1, c, TN2), lambda i, j: (i, 0, j)),
            pl.BlockSpec((c, 1), lambda i, j: (0, 0)),
            pl.BlockSpec((c, 1), lambda i, j: (0, 0)),
        ],
        out_specs=pl.BlockSpec((1, c, TN2), lambda i, j: (i, 0, j)),
        out_shape=jax.ShapeDtypeStruct((b, c, n), jnp.float32),
    )(y_raw, scale, shift)


def _fold_stats(part, cnt, g, bias):
    c = part.shape[1] // K
    sums = part.reshape(2, K, c).sum(axis=1)
    mean = sums[0] / cnt
    var = sums[1] / cnt - mean * mean
    scale = g / jnp.sqrt(var + EPS)
    shift = bias - mean * scale
    return jnp.tile(scale, K).reshape(-1, 1), jnp.tile(shift, K).reshape(-1, 1)


def kernel(xyz, new_xyz, feature, new_feature, wn_w1, wn_g1, wn_b1, wn_w2,
           wn_g2, wn_b2, wn_w3, wn_g3, wn_b3, fin_w, fin_g, fin_b):
    B, N, _ = xyz.shape
    new_xyz_t = jnp.transpose(new_xyz, (0, 2, 1))
    xyz_t = jnp.transpose(xyz, (0, 2, 1))
    xyzp = jnp.concatenate([xyz_t, jnp.zeros((B, 1, N), jnp.float32)], axis=1)
    xyz48 = jnp.tile(xyzp, (1, K, 1))
    tbl = jnp.concatenate([new_xyz_t, new_feature], axis=1)

    idx = _knn(xyz, new_xyz_t)
    gx, gf = _gather(tbl.reshape(B, -1), idx, tbl.shape[2])

    eye = jnp.eye(K, dtype=jnp.float32)
    w1p = jnp.concatenate([wn_w1, jnp.zeros((16, 1), jnp.float32)], axis=1)
    w1k = jnp.kron(eye, w1p)
    w2k = jnp.kron(eye, wn_w2)
    w3k = jnp.kron(eye, wn_w3)

    cnt = jnp.float32(B * N * K)
    s1 = _stats_pass(gx, xyz48, [w1k])
    sc1, t1 = _fold_stats(s1, cnt, wn_g1, wn_b1)
    w1s = w1k * sc1
    s2 = _stats_pass(gx, xyz48, [w1s, t1, w2k])
    sc2, t2 = _fold_stats(s2, cnt, wn_g2, wn_b2)
    w2s = w2k * sc2
    s3 = _stats_pass(gx, xyz48, [w1s, t1, w2s, t2, w3k])
    sc3, t3 = _fold_stats(s3, cnt, wn_g3, wn_b3)
    w3s = w3k * sc3

    perm = jnp.asarray([(r % 32) * K + r // 32 for r in range(512)], jnp.int32)
    fwp = jnp.concatenate([fin_w[:, :512][:, perm], fin_w[:, 512:]], axis=1)

    y_raw, sy = _main_pass(gx, xyz48, gf, feature, w1s, t1, w2s, t2, w3s, t3, fwp)
    cnt_y = jnp.float32(B * N)
    mean_y = sy[0] / cnt_y
    var_y = sy[1] / cnt_y - mean_y * mean_y
    fsc = (fin_g / jnp.sqrt(var_y + EPS)).reshape(-1, 1)
    fsh = (fin_b - mean_y * fsc[:, 0]).reshape(-1, 1)
    return _final_pass(y_raw, fsc, fsh)
